# baseline (speedup 1.0000x reference)
import dataclasses
import functools

import jax
import jax.numpy as jnp
from jax import lax
from jax.experimental import pallas as pl
from jax.experimental.pallas import tpu as pltpu
from jax.experimental.pallas import tpu_sc as plsc

EPS = 1e-08

B = 4096
D = 256
KN = 32
KS = 200
K = KN + KS
KPAD = 240
NC, NS = 2, 16
NW = NC * NS
ROWS_PW = B // NW
C0 = 128
C1 = K - C0
LANES = 16
DP = D // 2


def _dot_one(g, ovb, k):
    gv = plsc.bitcast(g[k, pl.ds(0, LANES)], jnp.bfloat16)
    acc = gv * ovb[0]
    for i in range(1, DP // LANES):
        gv = plsc.bitcast(g[k, pl.ds(i * LANES, LANES)], jnp.bfloat16)
        acc = acc + gv * ovb[i]
    a, b = plsc.unpack(acc, format=plsc.PackFormat.INTERLEAVED,
                       preferred_element_type=jnp.float32)
    return jnp.sum(a + b)


def _dot_chunk(g, ovb, dots_v, s, off, n):
    lane = lax.broadcasted_iota(jnp.int32, (LANES,), 0)

    @plsc.parallel_loop(0, n // LANES, unroll=4)
    def _(j):
        vec = jnp.zeros((LANES,), jnp.float32)
        for i in range(LANES):
            val = _dot_one(g, ovb, j * LANES + i)
            vec = jnp.where(lane == i, val, vec)
        dots_v[s, pl.ds(off + j * LANES, LANES)] = vec

    rem = n % LANES
    if rem:
        vec = jnp.zeros((LANES,), jnp.float32)
        for i in range(rem):
            val = _dot_one(g, ovb, n - rem + i)
            vec = jnp.where(lane == i, val, vec)
        dots_v[s, pl.ds(off + n - rem, LANES)] = vec


_SC_PARAMS = pltpu.CompilerParams()
if "needs_layout_passes" in pltpu.CompilerParams.__dataclass_fields__:
    _SC_PARAMS = dataclasses.replace(_SC_PARAMS, needs_layout_passes=False)


@functools.partial(
    pl.kernel,
    out_type=jax.ShapeDtypeStruct((B, KPAD), jnp.float32),
    mesh=plsc.VectorSubcoreMesh(core_axis_name="c", subcore_axis_name="s"),
    compiler_params=_SC_PARAMS,
    scratch_types=[
        pltpu.VMEM((ROWS_PW, K), jnp.int32),
        pltpu.VMEM((2, DP), jnp.int32),
        pltpu.VMEM((C0, DP), jnp.int32),
        pltpu.VMEM((C1, DP), jnp.int32),
        pltpu.VMEM((2, KPAD), jnp.float32),
        pltpu.SemaphoreType.DMA,
        pltpu.SemaphoreType.DMA,
        pltpu.SemaphoreType.DMA,
        pltpu.SemaphoreType.DMA,
        pltpu.SemaphoreType.DMA,
        pltpu.SemaphoreType.DMA,
    ],
)
def _dots_kernel(feat_hbm, idx_hbm, ov_hbm, out_hbm,
                 idx_slab, ov_v, g0, g1, dots_v,
                 sem_g0, sem_g1, sem_ov0, sem_ov1, sem_out0, sem_out1):
    wid = lax.axis_index("s") * NC + lax.axis_index("c")
    base = wid * ROWS_PW
    last = ROWS_PW - 1
    sem_ov = (sem_ov0, sem_ov1)
    sem_out = (sem_out0, sem_out1)

    pltpu.sync_copy(idx_hbm.at[pl.ds(base, ROWS_PW)], idx_slab)
    pltpu.async_copy(ov_hbm.at[base], ov_v.at[0], sem_ov0)
    pltpu.async_copy(ov_hbm.at[base + 1], ov_v.at[1], sem_ov1)
    pltpu.async_copy(feat_hbm.at[idx_slab.at[0, pl.ds(0, C0)]], g0, sem_g0)

    @pl.loop(0, ROWS_PW, step=2)
    def _(r0):
        for s in (0, 1):
            r = r0 + s
            pltpu.async_copy(
                feat_hbm.at[idx_slab.at[r, pl.ds(C0, C1)]], g1, sem_g1)
            pltpu.make_async_copy(ov_hbm.at[base], ov_v.at[s],
                                  sem_ov[s]).wait()
            ovb = [plsc.bitcast(ov_v[s, pl.ds(i * LANES, LANES)],
                                jnp.bfloat16) for i in range(DP // LANES)]
            pltpu.make_async_copy(feat_hbm.at[pl.ds(0, C0)], g0,
                                  sem_g0).wait()

            @pl.when(r >= 2)
            def _():
                pltpu.make_async_copy(dots_v.at[s], out_hbm.at[base],
                                      sem_out[s]).wait()

            _dot_chunk(g0, ovb, dots_v, s, 0, C0)
            nxt = jnp.minimum(r + 1, last)
            pltpu.async_copy(
                feat_hbm.at[idx_slab.at[nxt, pl.ds(0, C0)]], g0, sem_g0)
            pltpu.make_async_copy(feat_hbm.at[pl.ds(0, C1)], g1,
                                  sem_g1).wait()
            _dot_chunk(g1, ovb, dots_v, s, C0, C1)
            pltpu.async_copy(dots_v.at[s], out_hbm.at[base + r], sem_out[s])
            nxt2 = jnp.minimum(r + 2, last)
            pltpu.async_copy(ov_hbm.at[base + nxt2], ov_v.at[s], sem_ov[s])

    pltpu.make_async_copy(feat_hbm.at[pl.ds(0, C0)], g0, sem_g0).wait()
    pltpu.make_async_copy(ov_hbm.at[base], ov_v.at[0], sem_ov0).wait()
    pltpu.make_async_copy(ov_hbm.at[base], ov_v.at[1], sem_ov1).wait()
    pltpu.make_async_copy(dots_v.at[0], out_hbm.at[base], sem_out0).wait()
    pltpu.make_async_copy(dots_v.at[1], out_hbm.at[base], sem_out1).wait()


def _loss_body(dots_ref, out_ref):
    dots = dots_ref[:, :K]
    m = jnp.max(dots, axis=1, keepdims=True)
    sh = dots - m
    dn = jnp.sum(sh[:, :KN], axis=1) / KN
    es = jnp.sum(jnp.exp(sh[:, KN:]), axis=1) / KS
    logits = jnp.log(EPS + jnp.exp(dn)) - jnp.log(EPS + es)
    out_ref[...] = jnp.reshape(-jnp.mean(logits), (1, 1))


def _loss(dots):
    return pl.pallas_call(
        _loss_body,
        out_shape=jax.ShapeDtypeStruct((1, 1), jnp.float32),
    )(dots)


def _pack_body(x_ref, out_ref):
    xb = x_ref[...].astype(jnp.bfloat16)
    lo = lax.bitcast_convert_type(xb[:, :DP], jnp.uint16).astype(jnp.uint32)
    hi = lax.bitcast_convert_type(xb[:, DP:], jnp.uint16).astype(jnp.uint32)
    out_ref[...] = lax.bitcast_convert_type(lo | (hi << 16), jnp.int32)


def _pack_bf16(x):
    n = x.shape[0]
    blk = 5000 if n % 5000 == 0 else 2048
    return pl.pallas_call(
        _pack_body,
        grid=(n // blk,),
        in_specs=[pl.BlockSpec((blk, D), lambda i: (i, 0))],
        out_specs=pl.BlockSpec((blk, DP), lambda i: (i, 0)),
        out_shape=jax.ShapeDtypeStruct((n, DP), jnp.int32),
    )(x)


_IDX_BLK = 512


def _idx_body(x_ref, s_ref, out_ref):
    out_ref[:, :KN] = x_ref[:, 1:1 + KN]
    out_ref[:, KN:] = s_ref[...]


def _build_idx(ix, smp):
    return pl.pallas_call(
        _idx_body,
        grid=(B // _IDX_BLK,),
        in_specs=[pl.BlockSpec((_IDX_BLK, KN + 1), lambda i: (i, 0)),
                  pl.BlockSpec((_IDX_BLK, KS), lambda i: (i, 0))],
        out_specs=pl.BlockSpec((_IDX_BLK, K), lambda i: (i, 0)),
        out_shape=jax.ShapeDtypeStruct((B, K), jnp.int32),
    )(ix, smp)


def kernel(features, mask, input_x, input_samples, output_vector):
    del mask
    idx = _build_idx(input_x.astype(jnp.int32),
                     input_samples.astype(jnp.int32))
    dots = _dots_kernel(_pack_bf16(features), idx, _pack_bf16(output_vector))
    return _loss(dots)[0, 0]


# speedup vs baseline: 1.5828x; 1.5828x over previous
import dataclasses
import functools

import jax
import jax.numpy as jnp
from jax import lax
from jax.experimental import pallas as pl
from jax.experimental.pallas import tpu as pltpu
from jax.experimental.pallas import tpu_sc as plsc

EPS = 1e-08

B = 4096
D = 256
KN = 32
KS = 200
K = KN + KS
KPAD = 240
NC, NS = 2, 16
NW = NC * NS
ROWS_PW = B // NW
C0 = 128
C1 = K - C0
LANES = 16
DP = D // 2


def _dot_one(g, ovb, k):
    gv = plsc.bitcast(g[k, pl.ds(0, LANES)], jnp.bfloat16)
    acc = gv * ovb[0]
    for i in range(1, DP // LANES):
        gv = plsc.bitcast(g[k, pl.ds(i * LANES, LANES)], jnp.bfloat16)
        acc = acc + gv * ovb[i]
    a, b = plsc.unpack(acc, format=plsc.PackFormat.INTERLEAVED,
                       preferred_element_type=jnp.float32)
    return jnp.sum(a + b)


def _dot_chunk(g, ovb, dots_v, s, off, n):
    lane = lax.broadcasted_iota(jnp.int32, (LANES,), 0)

    @plsc.parallel_loop(0, n // LANES, unroll=2)
    def _(j):
        vec = jnp.zeros((LANES,), jnp.float32)
        for i in range(LANES):
            val = _dot_one(g, ovb, j * LANES + i)
            vec = jnp.where(lane == i, val, vec)
        dots_v[s, pl.ds(off + j * LANES, LANES)] = vec

    rem = n % LANES
    if rem:
        vec = jnp.zeros((LANES,), jnp.float32)
        for i in range(rem):
            val = _dot_one(g, ovb, n - rem + i)
            vec = jnp.where(lane == i, val, vec)
        dots_v[s, pl.ds(off + n - rem, LANES)] = vec


_SC_PARAMS = pltpu.CompilerParams()
if "needs_layout_passes" in pltpu.CompilerParams.__dataclass_fields__:
    _SC_PARAMS = dataclasses.replace(_SC_PARAMS, needs_layout_passes=False)


@functools.partial(
    pl.kernel,
    out_type=jax.ShapeDtypeStruct((B, KPAD), jnp.float32),
    mesh=plsc.VectorSubcoreMesh(core_axis_name="c", subcore_axis_name="s"),
    compiler_params=_SC_PARAMS,
    scratch_types=[
        pltpu.VMEM((ROWS_PW, K), jnp.int32),
        pltpu.VMEM((2, DP), jnp.int32),
        pltpu.VMEM((C0, DP), jnp.int32),
        pltpu.VMEM((C1, DP), jnp.int32),
        pltpu.VMEM((2, KPAD), jnp.float32),
        pltpu.SemaphoreType.DMA,
        pltpu.SemaphoreType.DMA,
        pltpu.SemaphoreType.DMA,
        pltpu.SemaphoreType.DMA,
        pltpu.SemaphoreType.DMA,
        pltpu.SemaphoreType.DMA,
    ],
)
def _dots_kernel(feat_hbm, idx_hbm, ov_hbm, out_hbm,
                 idx_slab, ov_v, g0, g1, dots_v,
                 sem_g0, sem_g1, sem_ov0, sem_ov1, sem_out0, sem_out1):
    wid = lax.axis_index("s") * NC + lax.axis_index("c")
    base = wid * ROWS_PW
    last = ROWS_PW - 1
    sem_ov = (sem_ov0, sem_ov1)
    sem_out = (sem_out0, sem_out1)

    pltpu.sync_copy(idx_hbm.at[pl.ds(base, ROWS_PW)], idx_slab)
    pltpu.async_copy(ov_hbm.at[base], ov_v.at[0], sem_ov0)
    pltpu.async_copy(ov_hbm.at[base + 1], ov_v.at[1], sem_ov1)
    pltpu.async_copy(feat_hbm.at[idx_slab.at[0, pl.ds(0, C0)]], g0, sem_g0)

    @pl.loop(0, ROWS_PW, step=2)
    def _(r0):
        for s in (0, 1):
            r = r0 + s
            pltpu.async_copy(
                feat_hbm.at[idx_slab.at[r, pl.ds(C0, C1)]], g1, sem_g1)
            pltpu.make_async_copy(ov_hbm.at[base], ov_v.at[s],
                                  sem_ov[s]).wait()
            ovb = [plsc.bitcast(ov_v[s, pl.ds(i * LANES, LANES)],
                                jnp.bfloat16) for i in range(DP // LANES)]
            pltpu.make_async_copy(feat_hbm.at[pl.ds(0, C0)], g0,
                                  sem_g0).wait()

            @pl.when(r >= 2)
            def _():
                pltpu.make_async_copy(dots_v.at[s], out_hbm.at[base],
                                      sem_out[s]).wait()

            _dot_chunk(g0, ovb, dots_v, s, 0, C0)
            nxt = jnp.minimum(r + 1, last)
            pltpu.async_copy(
                feat_hbm.at[idx_slab.at[nxt, pl.ds(0, C0)]], g0, sem_g0)
            pltpu.make_async_copy(feat_hbm.at[pl.ds(0, C1)], g1,
                                  sem_g1).wait()
            _dot_chunk(g1, ovb, dots_v, s, C0, C1)
            pltpu.async_copy(dots_v.at[s], out_hbm.at[base + r], sem_out[s])
            nxt2 = jnp.minimum(r + 2, last)
            pltpu.async_copy(ov_hbm.at[base + nxt2], ov_v.at[s], sem_ov[s])

    pltpu.make_async_copy(feat_hbm.at[pl.ds(0, C0)], g0, sem_g0).wait()
    pltpu.make_async_copy(ov_hbm.at[base], ov_v.at[0], sem_ov0).wait()
    pltpu.make_async_copy(ov_hbm.at[base], ov_v.at[1], sem_ov1).wait()
    pltpu.make_async_copy(dots_v.at[0], out_hbm.at[base], sem_out0).wait()
    pltpu.make_async_copy(dots_v.at[1], out_hbm.at[base], sem_out1).wait()


def _loss_body(dots_ref, out_ref):
    dots = dots_ref[:, :K]
    m = jnp.max(dots, axis=1, keepdims=True)
    sh = dots - m
    dn = jnp.sum(sh[:, :KN], axis=1) / KN
    es = jnp.sum(jnp.exp(sh[:, KN:]), axis=1) / KS
    logits = jnp.log(EPS + jnp.exp(dn)) - jnp.log(EPS + es)
    out_ref[...] = jnp.reshape(-jnp.mean(logits), (1, 1))


def _loss(dots):
    return pl.pallas_call(
        _loss_body,
        out_shape=jax.ShapeDtypeStruct((1, 1), jnp.float32),
    )(dots)


def _pack_body(x_ref, out_ref):
    xb = x_ref[...].astype(jnp.bfloat16)
    lo = lax.bitcast_convert_type(xb[:, :DP], jnp.uint16).astype(jnp.uint32)
    hi = lax.bitcast_convert_type(xb[:, DP:], jnp.uint16).astype(jnp.uint32)
    out_ref[...] = lax.bitcast_convert_type(lo | (hi << 16), jnp.int32)


def _pack_bf16(x):
    n = x.shape[0]
    blk = 5000 if n % 5000 == 0 else 2048
    return pl.pallas_call(
        _pack_body,
        grid=(n // blk,),
        in_specs=[pl.BlockSpec((blk, D), lambda i: (i, 0))],
        out_specs=pl.BlockSpec((blk, DP), lambda i: (i, 0)),
        out_shape=jax.ShapeDtypeStruct((n, DP), jnp.int32),
    )(x)


_IDX_BLK = 512


def _idx_body(x_ref, s_ref, out_ref):
    out_ref[:, :KN] = x_ref[:, 1:1 + KN]
    out_ref[:, KN:] = s_ref[...]


def _build_idx(ix, smp):
    return pl.pallas_call(
        _idx_body,
        grid=(B // _IDX_BLK,),
        in_specs=[pl.BlockSpec((_IDX_BLK, KN + 1), lambda i: (i, 0)),
                  pl.BlockSpec((_IDX_BLK, KS), lambda i: (i, 0))],
        out_specs=pl.BlockSpec((_IDX_BLK, K), lambda i: (i, 0)),
        out_shape=jax.ShapeDtypeStruct((B, K), jnp.int32),
    )(ix, smp)


def kernel(features, mask, input_x, input_samples, output_vector):
    del mask
    idx = _build_idx(input_x.astype(jnp.int32),
                     input_samples.astype(jnp.int32))
    dots = _dots_kernel(_pack_bf16(features), idx, _pack_bf16(output_vector))
    return _loss(dots)[0, 0]


# speedup vs baseline: 1.9189x; 1.2123x over previous
import dataclasses
import functools

import jax
import jax.numpy as jnp
from jax import lax
from jax.experimental import pallas as pl
from jax.experimental.pallas import tpu as pltpu
from jax.experimental.pallas import tpu_sc as plsc

EPS = 1e-08

B = 4096
D = 256
KN = 32
KS = 200
K = KN + KS
KPAD = 240
NC, NS = 2, 16
NW = NC * NS
ROWS_PW = B // NW
C0 = 128
C1 = K - C0
LANES = 16
DP = D // 2


def _dot_one(g, ovb, k):
    gv = plsc.bitcast(g[k, pl.ds(0, LANES)], jnp.bfloat16)
    acc = gv * ovb[0]
    for i in range(1, DP // LANES):
        gv = plsc.bitcast(g[k, pl.ds(i * LANES, LANES)], jnp.bfloat16)
        acc = acc + gv * ovb[i]
    a, b = plsc.unpack(acc, format=plsc.PackFormat.INTERLEAVED,
                       preferred_element_type=jnp.float32)
    return jnp.sum(a + b)


def _dot_chunk(g, ovb, dots_v, s, off, n):
    lane = lax.broadcasted_iota(jnp.int32, (LANES,), 0)

    @plsc.parallel_loop(0, n // LANES, unroll=2)
    def _(j):
        vec = jnp.zeros((LANES,), jnp.float32)
        for i in range(LANES):
            val = _dot_one(g, ovb, j * LANES + i)
            vec = jnp.where(lane == i, val, vec)
        dots_v[s, pl.ds(off + j * LANES, LANES)] = vec

    rem = n % LANES
    if rem:
        vec = jnp.zeros((LANES,), jnp.float32)
        for i in range(rem):
            val = _dot_one(g, ovb, n - rem + i)
            vec = jnp.where(lane == i, val, vec)
        dots_v[s, pl.ds(off + n - rem, LANES)] = vec


_SC_PARAMS = pltpu.CompilerParams()
if "needs_layout_passes" in pltpu.CompilerParams.__dataclass_fields__:
    _SC_PARAMS = dataclasses.replace(_SC_PARAMS, needs_layout_passes=False)


@functools.partial(
    pl.kernel,
    out_type=jax.ShapeDtypeStruct((B, KPAD), jnp.float32),
    mesh=plsc.VectorSubcoreMesh(core_axis_name="c", subcore_axis_name="s"),
    compiler_params=_SC_PARAMS,
    scratch_types=[
        pltpu.VMEM((ROWS_PW, K), jnp.int32),
        pltpu.VMEM((2, DP), jnp.int32),
        pltpu.VMEM((2, C0, DP), jnp.int32),
        pltpu.VMEM((2, C1, DP), jnp.int32),
        pltpu.VMEM((2, KPAD), jnp.float32),
        pltpu.SemaphoreType.DMA,
        pltpu.SemaphoreType.DMA,
        pltpu.SemaphoreType.DMA,
        pltpu.SemaphoreType.DMA,
        pltpu.SemaphoreType.DMA,
        pltpu.SemaphoreType.DMA,
        pltpu.SemaphoreType.DMA,
        pltpu.SemaphoreType.DMA,
    ],
)
def _dots_kernel(feat_hbm, idx_hbm, ov_hbm, out_hbm,
                 idx_slab, ov_v, g0, g1, dots_v,
                 sem_g00, sem_g01, sem_g10, sem_g11,
                 sem_ov0, sem_ov1, sem_out0, sem_out1):
    wid = lax.axis_index("s") * NC + lax.axis_index("c")
    base = wid * ROWS_PW
    last = ROWS_PW - 1
    sem_g0 = (sem_g00, sem_g01)
    sem_g1 = (sem_g10, sem_g11)
    sem_ov = (sem_ov0, sem_ov1)
    sem_out = (sem_out0, sem_out1)

    pltpu.sync_copy(idx_hbm.at[pl.ds(base, ROWS_PW)], idx_slab)
    pltpu.async_copy(ov_hbm.at[base], ov_v.at[0], sem_ov0)
    pltpu.async_copy(ov_hbm.at[base + 1], ov_v.at[1], sem_ov1)
    pltpu.async_copy(feat_hbm.at[idx_slab.at[0, pl.ds(0, C0)]],
                     g0.at[0], sem_g00)
    pltpu.async_copy(feat_hbm.at[idx_slab.at[0, pl.ds(C0, C1)]],
                     g1.at[0], sem_g10)

    @pl.loop(0, ROWS_PW, step=2)
    def _(r0):
        for s in (0, 1):
            r = r0 + s
            t = 1 - s
            nxt = jnp.minimum(r + 1, last)
            pltpu.async_copy(
                feat_hbm.at[idx_slab.at[nxt, pl.ds(0, C0)]],
                g0.at[t], sem_g0[t])
            pltpu.make_async_copy(ov_hbm.at[base], ov_v.at[s],
                                  sem_ov[s]).wait()
            ovb = [plsc.bitcast(ov_v[s, pl.ds(i * LANES, LANES)],
                                jnp.bfloat16) for i in range(DP // LANES)]
            pltpu.make_async_copy(feat_hbm.at[pl.ds(0, C0)], g0.at[s],
                                  sem_g0[s]).wait()

            @pl.when(r >= 2)
            def _():
                pltpu.make_async_copy(dots_v.at[s], out_hbm.at[base],
                                      sem_out[s]).wait()

            _dot_chunk(g0.at[s], ovb, dots_v, s, 0, C0)
            pltpu.async_copy(
                feat_hbm.at[idx_slab.at[nxt, pl.ds(C0, C1)]],
                g1.at[t], sem_g1[t])
            pltpu.make_async_copy(feat_hbm.at[pl.ds(0, C1)], g1.at[s],
                                  sem_g1[s]).wait()
            _dot_chunk(g1.at[s], ovb, dots_v, s, C0, C1)
            pltpu.async_copy(dots_v.at[s], out_hbm.at[base + r], sem_out[s])
            nxt2 = jnp.minimum(r + 2, last)
            pltpu.async_copy(ov_hbm.at[base + nxt2], ov_v.at[s], sem_ov[s])

    pltpu.make_async_copy(feat_hbm.at[pl.ds(0, C0)], g0.at[0], sem_g00).wait()
    pltpu.make_async_copy(feat_hbm.at[pl.ds(0, C1)], g1.at[0], sem_g10).wait()
    pltpu.make_async_copy(ov_hbm.at[base], ov_v.at[0], sem_ov0).wait()
    pltpu.make_async_copy(ov_hbm.at[base], ov_v.at[1], sem_ov1).wait()
    pltpu.make_async_copy(dots_v.at[0], out_hbm.at[base], sem_out0).wait()
    pltpu.make_async_copy(dots_v.at[1], out_hbm.at[base], sem_out1).wait()


def _loss_body(dots_ref, out_ref):
    dots = dots_ref[:, :K]
    m = jnp.max(dots, axis=1, keepdims=True)
    sh = dots - m
    dn = jnp.sum(sh[:, :KN], axis=1) / KN
    es = jnp.sum(jnp.exp(sh[:, KN:]), axis=1) / KS
    logits = jnp.log(EPS + jnp.exp(dn)) - jnp.log(EPS + es)
    out_ref[...] = jnp.reshape(-jnp.mean(logits), (1, 1))


def _loss(dots):
    return pl.pallas_call(
        _loss_body,
        out_shape=jax.ShapeDtypeStruct((1, 1), jnp.float32),
    )(dots)


def _pack_body(x_ref, out_ref):
    xb = x_ref[...].astype(jnp.bfloat16)
    lo = lax.bitcast_convert_type(xb[:, :DP], jnp.uint16).astype(jnp.uint32)
    hi = lax.bitcast_convert_type(xb[:, DP:], jnp.uint16).astype(jnp.uint32)
    out_ref[...] = lax.bitcast_convert_type(lo | (hi << 16), jnp.int32)


def _pack_bf16(x):
    n = x.shape[0]
    blk = 5000 if n % 5000 == 0 else 2048
    return pl.pallas_call(
        _pack_body,
        grid=(n // blk,),
        in_specs=[pl.BlockSpec((blk, D), lambda i: (i, 0))],
        out_specs=pl.BlockSpec((blk, DP), lambda i: (i, 0)),
        out_shape=jax.ShapeDtypeStruct((n, DP), jnp.int32),
    )(x)


_IDX_BLK = 512


def _idx_body(x_ref, s_ref, out_ref):
    out_ref[:, :KN] = x_ref[:, 1:1 + KN]
    out_ref[:, KN:] = s_ref[...]


def _build_idx(ix, smp):
    return pl.pallas_call(
        _idx_body,
        grid=(B // _IDX_BLK,),
        in_specs=[pl.BlockSpec((_IDX_BLK, KN + 1), lambda i: (i, 0)),
                  pl.BlockSpec((_IDX_BLK, KS), lambda i: (i, 0))],
        out_specs=pl.BlockSpec((_IDX_BLK, K), lambda i: (i, 0)),
        out_shape=jax.ShapeDtypeStruct((B, K), jnp.int32),
    )(ix, smp)


def kernel(features, mask, input_x, input_samples, output_vector):
    del mask
    idx = _build_idx(input_x.astype(jnp.int32),
                     input_samples.astype(jnp.int32))
    dots = _dots_kernel(_pack_bf16(features), idx, _pack_bf16(output_vector))
    return _loss(dots)[0, 0]
